# baseline (speedup 1.0000x reference)
import jax
import jax.numpy as jnp
from jax import lax
from jax.experimental import pallas as pl
from jax.experimental.pallas import tpu as pltpu
from jax.experimental.pallas import tpu_sc as plsc

NC = 2
NS = 16
NW = NC * NS
L = 16

VOCAB = 1000000
DIM = 64
BATCH = 16384
NUM_CTX = 5
WIDE = 2 * DIM

CHUNK = 128
ROWS_PER_W = BATCH // NW
N_CHUNKS = ROWS_PER_W // CHUNK

GRP = 16
N_GRP = CHUNK // GRP
PAIRS = GRP * NUM_CTX


def _sc_kernel(sg_wide, ctx_wide, tgt_sup, tgt_half, ctx_sup, ctx_half, out,
               tgt_idx, tgt_hlf, ctx_idx, ctx_hlf, tgt_rows, ctx_rows,
               prods, out_v, sem):
  wid = lax.axis_index("s") * NC + lax.axis_index("c")
  iota = lax.iota(jnp.int32, L)
  gather_base = iota * L

  for c in range(N_CHUNKS):
    base = wid * ROWS_PER_W + c * CHUNK

    pltpu.sync_copy(tgt_sup.at[pl.ds(base, CHUNK)], tgt_idx)
    pltpu.sync_copy(tgt_half.at[pl.ds(base, CHUNK)], tgt_hlf)
    for j in range(NUM_CTX):
      pltpu.sync_copy(ctx_sup.at[pl.ds(j * BATCH + base, CHUNK)],
                      ctx_idx.at[pl.ds(j * CHUNK, CHUNK)])
    pltpu.sync_copy(ctx_half.at[pl.ds(base * NUM_CTX, CHUNK * NUM_CTX)],
                    ctx_hlf)

    copies = [pltpu.async_copy(sg_wide.at[tgt_idx], tgt_rows, sem)]
    for j in range(NUM_CTX):
      copies.append(
          pltpu.async_copy(ctx_wide.at[ctx_idx.at[pl.ds(j * CHUNK, CHUNK)]],
                           ctx_rows.at[pl.ds(j * CHUNK, CHUNK)], sem))
    for cp in copies:
      cp.wait()

    def body(g, carry):
      b0 = g * GRP
      thv = tgt_hlf[pl.ds(b0, L)]
      chv = [ctx_hlf[pl.ds(b0 * NUM_CTX + q * L, L)]
             for q in range(PAIRS // L)]
      for bi in range(GRP):
        b = b0 + bi
        tho = thv[bi]
        tcols = [iota + (tho + k * L) for k in range(DIM // L)]
        brow = jnp.full((L,), b, jnp.int32)
        tb = [plsc.load_gather(tgt_rows, [brow, tcols[k]])
              for k in range(DIM // L)]
        for j in range(NUM_CTX):
          r = j * CHUNK + b
          p = bi * NUM_CTX + j
          cho = chv[p // L][p % L]
          rrow = jnp.full((L,), r, jnp.int32)
          ccols = iota + cho
          acc = plsc.load_gather(ctx_rows, [rrow, ccols]) * tb[0]
          for k in range(1, DIM // L):
            acc = acc + plsc.load_gather(
                ctx_rows, [rrow, ccols + k * L]) * tb[k]
          prods[pl.ds(p * L, L)] = acc
      for o in range(PAIRS // L):
        sums = plsc.load_gather(prods, [gather_base + o * (L * L)])
        for k in range(1, L):
          sums = sums + plsc.load_gather(
              prods, [gather_base + (o * (L * L) + k)])
        out_v[pl.ds(b0 * NUM_CTX + o * L, L)] = sums
      return carry

    lax.fori_loop(0, N_GRP, body, 0)

    pltpu.sync_copy(out_v, out.at[pl.ds(base * NUM_CTX, CHUNK * NUM_CTX)])


@jax.jit
def _run(tgt_sup, tgt_half, ctx_sup, ctx_half, sg_wide, ctx_wide):
  mesh = plsc.VectorSubcoreMesh(core_axis_name="c", subcore_axis_name="s")
  return pl.kernel(
      _sc_kernel,
      out_type=jax.ShapeDtypeStruct((BATCH * NUM_CTX,), jnp.float32),
      mesh=mesh,
      compiler_params=pltpu.CompilerParams(needs_layout_passes=False),
      scratch_types=[
          pltpu.VMEM((CHUNK,), jnp.int32),
          pltpu.VMEM((CHUNK,), jnp.int32),
          pltpu.VMEM((NUM_CTX * CHUNK,), jnp.int32),
          pltpu.VMEM((NUM_CTX * CHUNK,), jnp.int32),
          pltpu.VMEM((CHUNK, WIDE), jnp.float32),
          pltpu.VMEM((NUM_CTX * CHUNK, WIDE), jnp.float32),
          pltpu.VMEM((PAIRS * L,), jnp.float32),
          pltpu.VMEM((CHUNK * NUM_CTX,), jnp.float32),
          pltpu.SemaphoreType.DMA,
      ],
  )(sg_wide, ctx_wide, tgt_sup, tgt_half, ctx_sup, ctx_half)


def kernel(target, context, sg_table, ctx_table):
  target = target.astype(jnp.int32)
  context = context.astype(jnp.int32)
  sg_wide = sg_table.reshape(VOCAB // 2, WIDE)
  ctx_wide = ctx_table.reshape(VOCAB // 2, WIDE)
  tgt_sup = target >> 1
  tgt_half = (target & 1) * DIM
  ctx_sup = jnp.transpose(context >> 1, (1, 0)).reshape(-1)
  ctx_half = ((context & 1) * DIM).reshape(-1)
  out_flat = _run(tgt_sup, tgt_half, ctx_sup, ctx_half, sg_wide, ctx_wide)
  return out_flat.reshape(BATCH, NUM_CTX)
